# baseline (speedup 1.0000x reference)
import jax
import jax.numpy as jnp
from jax import lax
from jax.experimental import pallas as pl
from jax.experimental.pallas import tpu as pltpu
from jax.experimental.pallas import tpu_sc as plsc

BATCH = 16384
HIST_LEN = 200
FEATURES = 64
N = BATCH * HIST_LEN

NUM_CORES = 2
NUM_SUBCORES = 16
CHUNK = 256
NITER = N // (NUM_CORES * NUM_SUBCORES * CHUNK)
L = 16
NQ = FEATURES // L
RB = 16
K = 3

CW = CHUNK * FEATURES
SLOT = NUM_SUBCORES * CW


def _body(x_hbm, tab_hbm, out_hbm, x_v, tab_v, rows_v, sp, sem_x, sem_s,
          sem_d):
    c = lax.axis_index("c")
    s = lax.axis_index("s")

    def xbase(i):
        return ((c * NITER + i) * NUM_SUBCORES + s) * CHUNK

    def x_copy(i, bx):
        return pltpu.make_async_copy(
            x_hbm.at[pl.ds(xbase(i), CHUNK)],
            x_v.at[pl.ds(bx * CHUNK, CHUNK)], sem_x)

    def stream_rows(i, br):
        slot = lax.rem(i, K)
        return pltpu.make_async_copy(
            rows_v.at[pl.ds(br * CW, CW)],
            sp.at[pl.ds(slot * SLOT + s * CW, CW)], sem_s)

    def dma_out(i):
        slot = lax.rem(i, K)
        obase = (c * NITER + i) * SLOT
        return pltpu.make_async_copy(
            sp.at[pl.ds(slot * SLOT, SLOT)], out_hbm.at[pl.ds(obase, SLOT)],
            sem_d.at[slot])

    pltpu.sync_copy(tab_hbm, tab_v)
    w0 = [tab_v[0, pl.ds(q * L, L)] for q in range(NQ)]
    w1 = [tab_v[1, pl.ds(q * L, L)] for q in range(NQ)]
    one = jnp.full((L,), 1, jnp.int32)

    x_copy(0, 0).start()

    def step(i, carry):
        b = lax.rem(i, 2)

        x_copy(i, b).wait()

        @pl.when(i + 1 < NITER)
        def _():
            x_copy(i + 1, 1 - b).start()

        @pl.when(i >= 1)
        def _():
            stream_rows(i - 1, 1 - b).wait()

        @pl.when((i >= K) & (s == 0))
        def _():
            dma_out(i - K).wait()

        plsc.subcore_barrier()

        @pl.when((i >= 1) & (s == 0))
        def _():
            dma_out(i - 1).start()

        def block(j, carry2):
            rbase = b * CW + j * RB * FEATURES
            xv = x_v[pl.ds(b * CHUNK + j * RB, L)]
            for t in range(RB):
                m = jnp.full((L,), xv[t], jnp.int32) == one
                for q in range(NQ):
                    off = pl.multiple_of(rbase + t * FEATURES + q * L, L)
                    rows_v[pl.ds(off, L)] = jnp.where(m, w1[q], w0[q])
            return carry2

        lax.fori_loop(0, CHUNK // RB, block, 0)

        stream_rows(i, b).start()
        return carry

    lax.fori_loop(0, NITER, step, 0)

    last = NITER - 1
    stream_rows(last, lax.rem(last, 2)).wait()
    plsc.subcore_barrier()

    @pl.when(s == 0)
    def _():
        dma_out(last).start()
        for k in range(K):
            dma_out(last - k).wait()

    plsc.subcore_barrier()


@jax.jit
def _lookup(x_flat, table):
    f = pl.kernel(
        _body,
        out_type=jax.ShapeDtypeStruct((N * FEATURES,), jnp.float32),
        mesh=plsc.VectorSubcoreMesh(core_axis_name="c", subcore_axis_name="s"),
        scratch_types=[
            pltpu.VMEM((2 * CHUNK,), jnp.int32),
            pltpu.VMEM((2, FEATURES), jnp.float32),
            pltpu.VMEM((2 * CW,), jnp.float32),
            pltpu.VMEM_SHARED((K * SLOT,), jnp.float32),
            pltpu.SemaphoreType.DMA,
            pltpu.SemaphoreType.DMA,
            pltpu.SemaphoreType.DMA((K,)),
        ],
        compiler_params=pltpu.CompilerParams(needs_layout_passes=False),
    )
    return f(x_flat, table)


def kernel(x, table):
    out = _lookup(x.reshape(N), table)
    return out.reshape(BATCH, HIST_LEN, FEATURES)


# speedup vs baseline: 1.1205x; 1.1205x over previous
import jax
import jax.numpy as jnp
from jax import lax
from jax.experimental import pallas as pl
from jax.experimental.pallas import tpu as pltpu
from jax.experimental.pallas import tpu_sc as plsc

BATCH = 16384
HIST_LEN = 200
FEATURES = 64
N = BATCH * HIST_LEN

NUM_CORES = 2
NUM_SUBCORES = 16
CHUNK = 512
NITER = N // (NUM_CORES * NUM_SUBCORES * CHUNK)
L = 16
NQ = FEATURES // L
RB = 16
K = 3

CW = CHUNK * FEATURES


def _body(x_hbm, tab_hbm, out_hbm, x_v, tab_v, rows_v, sem_x, sem_o):
    c = lax.axis_index("c")
    s = lax.axis_index("s")
    wbase = (c * NUM_SUBCORES + s) * (NITER * CHUNK)

    def x_copy(i, bx):
        return pltpu.make_async_copy(
            x_hbm.at[pl.ds(wbase + i * CHUNK, CHUNK)],
            x_v.at[pl.ds(bx * CHUNK, CHUNK)], sem_x)

    def out_copy(i):
        slot = lax.rem(i, K)
        return pltpu.make_async_copy(
            rows_v.at[pl.ds(slot * CW, CW)],
            out_hbm.at[pl.ds((wbase + i * CHUNK) * FEATURES, CW)],
            sem_o.at[slot])

    pltpu.sync_copy(tab_hbm, tab_v)
    w0 = [tab_v[0, pl.ds(q * L, L)] for q in range(NQ)]
    w1 = [tab_v[1, pl.ds(q * L, L)] for q in range(NQ)]
    one = jnp.full((L,), 1, jnp.int32)

    x_copy(0, 0).start()

    def step(i, carry):
        b = lax.rem(i, 2)
        slot = lax.rem(i, K)

        x_copy(i, b).wait()

        @pl.when(i + 1 < NITER)
        def _():
            x_copy(i + 1, 1 - b).start()

        @pl.when(i >= K)
        def _():
            out_copy(i - K).wait()

        def block(j, carry2):
            rbase = slot * CW + j * RB * FEATURES
            xv = x_v[pl.ds(b * CHUNK + j * RB, L)]
            for t in range(RB):
                m = jnp.full((L,), xv[t], jnp.int32) == one
                for q in range(NQ):
                    off = pl.multiple_of(rbase + t * FEATURES + q * L, L)
                    rows_v[pl.ds(off, L)] = jnp.where(m, w1[q], w0[q])
            return carry2

        lax.fori_loop(0, CHUNK // RB, block, 0)

        out_copy(i).start()
        return carry

    lax.fori_loop(0, NITER, step, 0)

    for k in range(K):
        out_copy(NITER - 1 - k).wait()


@jax.jit
def _lookup(x_flat, table):
    f = pl.kernel(
        _body,
        out_type=jax.ShapeDtypeStruct((N * FEATURES,), jnp.float32),
        mesh=plsc.VectorSubcoreMesh(core_axis_name="c", subcore_axis_name="s"),
        scratch_types=[
            pltpu.VMEM((2 * CHUNK,), jnp.int32),
            pltpu.VMEM((2, FEATURES), jnp.float32),
            pltpu.VMEM((K * CW,), jnp.float32),
            pltpu.SemaphoreType.DMA,
            pltpu.SemaphoreType.DMA((K,)),
        ],
        compiler_params=pltpu.CompilerParams(needs_layout_passes=False),
    )
    return f(x_flat, table)


def kernel(x, table):
    out = _lookup(x.reshape(N), table)
    return out.reshape(BATCH, HIST_LEN, FEATURES)


# speedup vs baseline: 1.4806x; 1.3214x over previous
import jax
import jax.numpy as jnp
from jax import lax
from jax.experimental import pallas as pl
from jax.experimental.pallas import tpu as pltpu
from jax.experimental.pallas import tpu_sc as plsc

BATCH = 16384
HIST_LEN = 200
FEATURES = 64
N = BATCH * HIST_LEN

NUM_CORES = 2
NUM_SUBCORES = 16
WB = BATCH // (NUM_CORES * NUM_SUBCORES)
PB = 2
NITER = WB // PB
CHUNK = PB * HIST_LEN
L = 16
NQ = FEATURES // L
RB = 16
K = 2


def _body(x_hbm, tab_hbm, out_hbm, x_v, tab_v, rows_v, sem_x, sem_o):
    c = lax.axis_index("c")
    s = lax.axis_index("s")
    w = c * NUM_SUBCORES + s

    def x_copy(i, bx):
        return pltpu.make_async_copy(
            x_hbm.at[pl.ds((w * WB + i * PB) * HIST_LEN, CHUNK)],
            x_v.at[pl.ds(bx * CHUNK, CHUNK)], sem_x)

    def out_copy(i, p):
        slot = lax.rem(i, K)
        return pltpu.make_async_copy(
            rows_v.at[slot, pl.ds(p * HIST_LEN, HIST_LEN)],
            out_hbm.at[w * WB + i * PB + p],
            sem_o.at[slot * PB + p])

    pltpu.sync_copy(tab_hbm, tab_v)
    w0 = [tab_v[0, pl.ds(q * L, L)] for q in range(NQ)]
    w1 = [tab_v[1, pl.ds(q * L, L)] for q in range(NQ)]
    one = jnp.full((L,), 1, jnp.int32)

    x_copy(0, 0).start()

    def step(i, carry):
        b = lax.rem(i, 2)
        slot = lax.rem(i, K)

        x_copy(i, b).wait()

        @pl.when(i + 1 < NITER)
        def _():
            x_copy(i + 1, 1 - b).start()

        @pl.when(i >= K)
        def _():
            for p in range(PB):
                out_copy(i - K, p).wait()

        def block(j, carry2):
            xv = x_v[pl.ds(b * CHUNK + j * RB, L)]
            for t in range(RB):
                m = jnp.full((L,), xv[t], jnp.int32) == one
                for q in range(NQ):
                    rows_v[slot, j * RB + t, pl.ds(q * L, L)] = (
                        jnp.where(m, w1[q], w0[q]))
            return carry2

        lax.fori_loop(0, CHUNK // RB, block, 0)

        for p in range(PB):
            out_copy(i, p).start()
        return carry

    lax.fori_loop(0, NITER, step, 0)

    for k in range(K):
        for p in range(PB):
            out_copy(NITER - 1 - k, p).wait()


@jax.jit
def _lookup(x_flat, table):
    f = pl.kernel(
        _body,
        out_type=jax.ShapeDtypeStruct((BATCH, HIST_LEN, FEATURES),
                                      jnp.float32),
        mesh=plsc.VectorSubcoreMesh(core_axis_name="c", subcore_axis_name="s"),
        scratch_types=[
            pltpu.VMEM((2 * CHUNK,), jnp.int32),
            pltpu.VMEM((2, FEATURES), jnp.float32),
            pltpu.VMEM((K, CHUNK, FEATURES), jnp.float32),
            pltpu.SemaphoreType.DMA,
            pltpu.SemaphoreType.DMA((K * PB,)),
        ],
        compiler_params=pltpu.CompilerParams(
            needs_layout_passes=False, use_tc_tiling_on_sc=True),
    )
    return f(x_flat, table)


def kernel(x, table):
    return _lookup(x.reshape(N), table)
